# baseline (speedup 1.0000x reference)
import functools

import jax
import jax.numpy as jnp
from jax import lax
from jax.experimental import pallas as pl
from jax.experimental.pallas import tpu as pltpu
from jax.experimental.pallas import tpu_sc as plsc

B = 16384
N_FACTORS = 32
HIDDEN = 64
_PACK = 128 // N_FACTORS

_INFO = plsc.get_sparse_core_info()
_NC = _INFO.num_cores
_NS = _INFO.num_subcores
_NW = _NC * _NS
_BPW = B // _NW
_CHUNK = 128
_NCHUNK = _BPW // _CHUNK
_L = 16


def _shift_idx(idx_ref):
    for k in range(_CHUNK // _L):
        sl = pl.ds(k * _L, _L)
        idx_ref[sl] = lax.shift_right_logical(idx_ref[sl], 2)


def _sc_gather_body(user_hbm, movie_hbm, U_hbm, M_hbm, ue_hbm, me_hbm,
                    uidx0, uidx1, uidx2, uidx3,
                    midx0, midx1, midx2, midx3,
                    rows_v, sem):
    uidx = (uidx0, uidx1, uidx2, uidx3)
    midx = (midx0, midx1, midx2, midx3)
    wid = lax.axis_index("s") * _NC + lax.axis_index("c")
    base = wid * _BPW
    for j in range(_NCHUNK):
        pltpu.sync_copy(user_hbm.at[pl.ds(base + j * _CHUNK, _CHUNK)],
                        uidx[j])
        _shift_idx(uidx[j])
    ucopies = [
        pltpu.async_copy(U_hbm.at[uidx[j]],
                         rows_v.at[pl.ds(j * _CHUNK, _CHUNK)], sem)
        for j in range(_NCHUNK)
    ]
    for j in range(_NCHUNK):
        pltpu.sync_copy(movie_hbm.at[pl.ds(base + j * _CHUNK, _CHUNK)],
                        midx[j])
        _shift_idx(midx[j])
    for c in ucopies:
        c.wait()
    pltpu.sync_copy(rows_v, ue_hbm.at[pl.ds(base, _BPW)])
    mcopies = [
        pltpu.async_copy(M_hbm.at[midx[j]],
                         rows_v.at[pl.ds(j * _CHUNK, _CHUNK)], sem)
        for j in range(_NCHUNK)
    ]
    for c in mcopies:
        c.wait()
    pltpu.sync_copy(rows_v, me_hbm.at[pl.ds(base, _BPW)])


def _sc_gather(user, movie, Ur, Mr):
    mesh = plsc.VectorSubcoreMesh(core_axis_name="c", subcore_axis_name="s")
    f = functools.partial(
        pl.kernel, mesh=mesh,
        out_type=[
            jax.ShapeDtypeStruct((B, 128), jnp.float32),
            jax.ShapeDtypeStruct((B, 128), jnp.float32),
        ],
        scratch_types=(
            [pltpu.VMEM((_CHUNK,), jnp.int32) for _ in range(2 * _NCHUNK)]
            + [pltpu.VMEM((_BPW, 128), jnp.float32),
               pltpu.SemaphoreType.DMA]
        ),
    )(_sc_gather_body)
    return f(user, movie, Ur, Mr)


def _mlp_body(uep_ref, mep_ref, user_ref, movie_ref,
              w1a_ref, w1b_ref, b1_ref, w2_ref, b2_ref, out_ref):
    usel = lax.rem(user_ref[...], _PACK)
    msel = lax.rem(movie_ref[...], _PACK)
    uep = uep_ref[...]
    mep = mep_ref[...]
    ue = jnp.zeros(uep.shape[:1] + (N_FACTORS,), jnp.float32)
    me = jnp.zeros_like(ue)
    for g in range(_PACK):
        sl = slice(g * N_FACTORS, (g + 1) * N_FACTORS)
        ue = ue + jnp.where(usel == g, uep[:, sl], 0.0)
        me = me + jnp.where(msel == g, mep[:, sl], 0.0)
    h = jnp.dot(ue, w1a_ref[...], preferred_element_type=jnp.float32)
    h = h + jnp.dot(me, w1b_ref[...], preferred_element_type=jnp.float32)
    h = jnp.maximum(h + b1_ref[...], 0.0)
    y = jnp.dot(h, w2_ref[...], preferred_element_type=jnp.float32)
    y = y + b2_ref[...]
    out_ref[...] = jax.nn.sigmoid(y) * 5.5


def _tc_mlp(ue_pad, me_pad, user2, movie2, W1, b1, W2, b2):
    bm = 2048
    grid = (B // bm,)
    w1a = W1[:N_FACTORS]
    w1b = W1[N_FACTORS:]
    b1r = b1.reshape(1, HIDDEN)
    b2r = b2.reshape(1, 1)
    return pl.pallas_call(
        _mlp_body,
        grid=grid,
        in_specs=[
            pl.BlockSpec((bm, 128), lambda i: (i, 0)),
            pl.BlockSpec((bm, 128), lambda i: (i, 0)),
            pl.BlockSpec((bm, 1), lambda i: (i, 0)),
            pl.BlockSpec((bm, 1), lambda i: (i, 0)),
            pl.BlockSpec((N_FACTORS, HIDDEN), lambda i: (0, 0)),
            pl.BlockSpec((N_FACTORS, HIDDEN), lambda i: (0, 0)),
            pl.BlockSpec((1, HIDDEN), lambda i: (0, 0)),
            pl.BlockSpec((HIDDEN, 1), lambda i: (0, 0)),
            pl.BlockSpec((1, 1), lambda i: (0, 0)),
        ],
        out_specs=pl.BlockSpec((bm, 1), lambda i: (i, 0)),
        out_shape=jax.ShapeDtypeStruct((B, 1), jnp.float32),
    )(ue_pad, me_pad, user2, movie2, w1a, w1b, b1r, W2, b2r)


def kernel(user, movie, U, M, W1, b1, W2, b2):
    user = user.astype(jnp.int32)
    movie = movie.astype(jnp.int32)
    Ur = U.reshape(U.shape[0] // _PACK, 128)
    Mr = M.reshape(M.shape[0] // _PACK, 128)
    ue_pad, me_pad = _sc_gather(user, movie, Ur, Mr)
    return _tc_mlp(ue_pad, me_pad, user.reshape(B, 1), movie.reshape(B, 1),
                   W1, b1, W2, b2)


# speedup vs baseline: 2.0037x; 2.0037x over previous
import functools

import jax
import jax.numpy as jnp
from jax import lax
from jax.experimental import pallas as pl
from jax.experimental.pallas import tpu as pltpu
from jax.experimental.pallas import tpu_sc as plsc

B = 16384
N_FACTORS = 32
HIDDEN = 64
N_USERS = 1000000
N_MOVIES = 100000

_INFO = plsc.get_sparse_core_info()
_NC = _INFO.num_cores
_NS = _INFO.num_subcores
_NW = _NC * _NS
_BPW = B // _NW
_L = 16
_G = _BPW // _L

_U_LAST_TILE = (N_USERS - 128) // 128
_M_LAST_TILE = (N_MOVIES - 128) // 128
_U_TAIL0 = (_U_LAST_TILE + 1) * 128
_M_TAIL0 = (_M_LAST_TILE + 1) * 128
_U_TAIL = N_USERS - _U_TAIL0
_M_TAIL = N_MOVIES - _M_TAIL0


def _gather_phase(idx_hbm, tbl_hbm, out_hbm, idx_v, tiles_v, ob_v, sem,
                  base, last_tile):
    pltpu.sync_copy(idx_hbm.at[pl.ds(base, _BPW)], idx_v)
    rows_lo = lax.iota(jnp.int32, _L)
    rows_hi = rows_lo + _L

    def group(g, carry):
        vec = idx_v[pl.ds(g * _L, _L)]
        tile_ids = jnp.minimum(lax.shift_right_logical(vec, 7),
                               jnp.int32(last_tile))
        copies = []
        for j in range(_L):
            t = pl.multiple_of(tile_ids[j] * 128, 128)
            copies.append(pltpu.async_copy(
                tbl_hbm.at[:, pl.ds(t, 128)], tiles_v.at[j], sem))
        for c in copies:
            c.wait()
        lanes = vec - tile_ids * 128
        for j in range(_L):
            lane = lanes[j]
            cols = jnp.broadcast_to(lane, (_L,))
            g0 = plsc.load_gather(tiles_v.at[j], [rows_lo, cols])
            g1 = plsc.load_gather(tiles_v.at[j], [rows_hi, cols])
            ob_v[j, pl.ds(0, _L)] = g0
            ob_v[j, pl.ds(_L, _L)] = g1
        pltpu.sync_copy(ob_v, out_hbm.at[pl.ds(base + g * _L, _L), :])
        return carry

    lax.fori_loop(0, _G, group, jnp.int32(0))


def _sc_body(user_hbm, movie_hbm, Ut_hbm, Mt_hbm, uout_hbm, mout_hbm,
             idx_v, tiles_v, ob_v, sem):
    wid = lax.axis_index("s") * _NC + lax.axis_index("c")
    base = wid * _BPW
    _gather_phase(user_hbm, Ut_hbm, uout_hbm, idx_v, tiles_v, ob_v, sem,
                  base, _U_LAST_TILE)
    _gather_phase(movie_hbm, Mt_hbm, mout_hbm, idx_v, tiles_v, ob_v, sem,
                  base, _M_LAST_TILE)


def _sc_gather(user, movie, Ut, Mt):
    mesh = plsc.VectorSubcoreMesh(core_axis_name="c", subcore_axis_name="s")
    f = functools.partial(
        pl.kernel, mesh=mesh,
        compiler_params=pltpu.CompilerParams(needs_layout_passes=False),
        out_type=[
            jax.ShapeDtypeStruct((B, 128), jnp.float32),
            jax.ShapeDtypeStruct((B, 128), jnp.float32),
        ],
        scratch_types=[
            pltpu.VMEM((_BPW,), jnp.int32),
            pltpu.VMEM((_L, N_FACTORS, 128), jnp.float32),
            pltpu.VMEM((_L, 128), jnp.float32),
            pltpu.SemaphoreType.DMA,
        ],
    )(_sc_body)
    return f(user, movie, Ut, Mt)


def _mlp_body(uep_ref, mep_ref, user_ref, movie_ref, tailu_ref, tailm_ref,
              w1a_ref, w1b_ref, b1_ref, w2_ref, b2_ref, out_ref):
    ue = uep_ref[:, :N_FACTORS]
    me = mep_ref[:, :N_FACTORS]
    user = user_ref[...]
    movie = movie_ref[...]
    du = user - _U_TAIL0
    ohu = (du == lax.broadcasted_iota(jnp.int32, (1, _U_TAIL), 1)
           ).astype(jnp.float32)
    ue = jnp.where(user >= _U_TAIL0, 0.0, ue) + jnp.dot(
        ohu, tailu_ref[...], preferred_element_type=jnp.float32)
    dm = movie - _M_TAIL0
    ohm = (dm == lax.broadcasted_iota(jnp.int32, (1, _M_TAIL), 1)
           ).astype(jnp.float32)
    me = jnp.where(movie >= _M_TAIL0, 0.0, me) + jnp.dot(
        ohm, tailm_ref[...], preferred_element_type=jnp.float32)
    h = jnp.dot(ue, w1a_ref[...], preferred_element_type=jnp.float32)
    h = h + jnp.dot(me, w1b_ref[...], preferred_element_type=jnp.float32)
    h = jnp.maximum(h + b1_ref[...], 0.0)
    y = jnp.dot(h, w2_ref[...], preferred_element_type=jnp.float32)
    y = y + b2_ref[...]
    out_ref[...] = jax.nn.sigmoid(y) * 5.5


def _tc_mlp(ue_pad, me_pad, user2, movie2, tailU, tailM, W1, b1, W2, b2):
    bm = 2048
    grid = (B // bm,)
    w1a = W1[:N_FACTORS]
    w1b = W1[N_FACTORS:]
    b1r = b1.reshape(1, HIDDEN)
    b2r = b2.reshape(1, 1)
    return pl.pallas_call(
        _mlp_body,
        grid=grid,
        in_specs=[
            pl.BlockSpec((bm, 128), lambda i: (i, 0)),
            pl.BlockSpec((bm, 128), lambda i: (i, 0)),
            pl.BlockSpec((bm, 1), lambda i: (i, 0)),
            pl.BlockSpec((bm, 1), lambda i: (i, 0)),
            pl.BlockSpec((_U_TAIL, N_FACTORS), lambda i: (0, 0)),
            pl.BlockSpec((_M_TAIL, N_FACTORS), lambda i: (0, 0)),
            pl.BlockSpec((N_FACTORS, HIDDEN), lambda i: (0, 0)),
            pl.BlockSpec((N_FACTORS, HIDDEN), lambda i: (0, 0)),
            pl.BlockSpec((1, HIDDEN), lambda i: (0, 0)),
            pl.BlockSpec((HIDDEN, 1), lambda i: (0, 0)),
            pl.BlockSpec((1, 1), lambda i: (0, 0)),
        ],
        out_specs=pl.BlockSpec((bm, 1), lambda i: (i, 0)),
        out_shape=jax.ShapeDtypeStruct((B, 1), jnp.float32),
    )(ue_pad, me_pad, user2, movie2, tailU, tailM, w1a, w1b, b1r, W2, b2r)


def kernel(user, movie, U, M, W1, b1, W2, b2):
    user = user.astype(jnp.int32)
    movie = movie.astype(jnp.int32)
    ue_pad, me_pad = _sc_gather(user, movie, U.T, M.T)
    tailU = U[_U_TAIL0:]
    tailM = M[_M_TAIL0:]
    return _tc_mlp(ue_pad, me_pad, user.reshape(B, 1), movie.reshape(B, 1),
                   tailU, tailM, W1, b1, W2, b2)
